# baseline (speedup 1.0000x reference)
import functools

import jax
import jax.numpy as jnp
from jax import lax
from jax.experimental import pallas as pl
from jax.experimental.pallas import tpu as pltpu
from jax.experimental.pallas import tpu_sc as plsc

HIDDEN = 128
SEG = 10000
NC, NS = 2, 16
NW = NC * NS
CHUNK = 128
SEGP = 10240
ZCH = SEGP // (NS * CHUNK)
ROWS_PER_TILE = SEGP // NS


def _ln(x, g, b, eps=1e-5):
    mu = jnp.mean(x, axis=-1, keepdims=True)
    xc = x - mu
    var = jnp.mean(xc * xc, axis=-1, keepdims=True)
    return xc * lax.rsqrt(var + eps) * g + b


def _mlp_body(h_ref, a_ref, w1t_ref, w1b_ref, b1_ref, w2t_ref, g1_ref,
              bb1_ref, w3t_ref, g2_ref, bb2_ref, x_ref, m_ref):
    h = h_ref[...]
    t = jnp.dot(h, w1t_ref[...], preferred_element_type=jnp.float32)
    t = t + a_ref[...] * w1b_ref[...] + b1_ref[...]
    t = jnp.maximum(t, 0.0)
    x = jnp.dot(t, w2t_ref[...], preferred_element_type=jnp.float32)
    x = _ln(x, g1_ref[...], bb1_ref[...])
    x_ref[...] = x
    m = jnp.dot(x, w3t_ref[...], preferred_element_type=jnp.float32)
    m_ref[...] = _ln(m, g2_ref[...], bb2_ref[...])


def _mlp(h_val, assign, W1, b1, W2, ln1_g, ln1_b, W3, ln2_g, ln2_b):
    n = h_val.shape[0]
    blk = 1000
    grid = n // blk
    full = lambda r, c: pl.BlockSpec((r, c), lambda i: (0, 0))
    return pl.pallas_call(
        _mlp_body,
        grid=(grid,),
        in_specs=[
            pl.BlockSpec((blk, HIDDEN), lambda i: (i, 0)),
            pl.BlockSpec((blk, 1), lambda i: (i, 0)),
            full(HIDDEN, HIDDEN),
            full(1, HIDDEN),
            full(1, HIDDEN),
            full(HIDDEN, HIDDEN),
            full(1, HIDDEN),
            full(1, HIDDEN),
            full(HIDDEN, 2 * HIDDEN),
            full(1, 2 * HIDDEN),
            full(1, 2 * HIDDEN),
        ],
        out_specs=[
            pl.BlockSpec((blk, HIDDEN), lambda i: (i, 0)),
            pl.BlockSpec((blk, 2 * HIDDEN), lambda i: (i, 0)),
        ],
        out_shape=[
            jax.ShapeDtypeStruct((n, HIDDEN), jnp.float32),
            jax.ShapeDtypeStruct((n, 2 * HIDDEN), jnp.float32),
        ],
    )(
        h_val,
        assign.reshape(n, 1),
        W1[:, :HIDDEN].T,
        W1[:, HIDDEN].reshape(1, HIDDEN),
        b1.reshape(1, HIDDEN),
        W2.T,
        ln1_g.reshape(1, HIDDEN),
        ln1_b.reshape(1, HIDDEN),
        W3.T,
        ln2_g.reshape(1, 2 * HIDDEN),
        ln2_b.reshape(1, 2 * HIDDEN),
    )


def _make_edge_kernel(cpt0, cpt1, stage, passes):
    mesh = plsc.VectorSubcoreMesh(
        core_axis_name="c", subcore_axis_name="s", num_cores=NC,
        num_subcores=NS)

    @functools.partial(
        pl.kernel,
        mesh=mesh,
        out_type=jax.ShapeDtypeStruct((NC, SEGP, HIDDEN), jnp.float32),
        scratch_types=[
            pltpu.VMEM((stage, CHUNK), jnp.int32),
            pltpu.VMEM((stage, CHUNK), jnp.int32),
            pltpu.VMEM((2 * CHUNK, HIDDEN), jnp.float32),
            pltpu.VMEM_SHARED((SEGP, HIDDEN), jnp.float32),
            pltpu.SemaphoreType.DMA,
        ],
    )
    def edge_kernel(m_hbm, oi_hbm, ii_hbm, out_hbm, oi_v, ii_v, buf, acc_sh,
                    sg):
        cid = lax.axis_index("c")
        sid = lax.axis_index("s")
        my_cpt = jnp.where(cid == 0, cpt0, cpt1)
        base0 = jnp.where(cid == 0, sid * cpt0, NS * cpt0 + sid * cpt1)

        zeros16 = jnp.zeros((16,), jnp.float32)

        def zrow(i, _):
            for k in range(HIDDEN // 16):
                buf[i, pl.ds(k * 16, 16)] = zeros16
            return 0

        lax.fori_loop(0, 2 * CHUNK, zrow, 0)
        for z in range(ZCH):
            pltpu.sync_copy(
                buf.at[pl.ds(0, CHUNK)],
                acc_sh.at[pl.ds((sid * ZCH + z) * CHUNK, CHUNK)])
        plsc.subcore_barrier()

        def body(j, _):
            @pl.when(j < stage)
            def _():
                pltpu.async_copy(
                    m_hbm.at[oi_v.at[j]],
                    buf.at[pl.ds((j % 2) * CHUNK, CHUNK)], sg)

            @pl.when(j > 0)
            def _():
                jp = j - 1
                slot = jp % 2
                pltpu.make_async_copy(
                    m_hbm.at[oi_v.at[jp]],
                    buf.at[pl.ds(slot * CHUNK, CHUNK)], sg).wait()
                pltpu.sync_copy(
                    buf.at[pl.ds(slot * CHUNK, CHUNK)],
                    acc_sh.at[ii_v.at[jp]], add=True)

            return 0

        for h in range(passes):
            @pl.when(h * stage < my_cpt)
            def _():
                cbase = pl.multiple_of(base0 + h * stage, 8)
                pltpu.sync_copy(oi_hbm.at[pl.ds(cbase, stage)], oi_v)
                pltpu.sync_copy(ii_hbm.at[pl.ds(cbase, stage)], ii_v)
                lax.fori_loop(0, stage + 1, body, 0)
        plsc.subcore_barrier()

        pltpu.sync_copy(
            acc_sh.at[pl.ds(sid * ROWS_PER_TILE, ROWS_PER_TILE)],
            out_hbm.at[cid, pl.ds(sid * ROWS_PER_TILE, ROWS_PER_TILE)])

    return edge_kernel


def _comb_body(p_ref, o_ref):
    o_ref[...] = p_ref[0] + p_ref[1]


def _combine(part):
    blk = 2000
    return pl.pallas_call(
        _comb_body,
        grid=(SEG // blk,),
        in_specs=[pl.BlockSpec((NC, blk, HIDDEN), lambda i: (0, i, 0))],
        out_specs=pl.BlockSpec((blk, HIDDEN), lambda i: (i, 0)),
        out_shape=jax.ShapeDtypeStruct((SEG, HIDDEN), jnp.float32),
    )(part)


def kernel(h_val, assign, cst_edges, LE, num_val, num_cst,
           W1, b1, W2, ln1_g, ln1_b, W3, ln2_g, ln2_b):
    n = h_val.shape[0]
    E = cst_edges.shape[1]

    x_val, m = _mlp(h_val, assign, W1, b1, W2, ln1_g, ln1_b, W3, ln2_g, ln2_b)
    m2 = m.reshape(2 * n, HIDDEN)

    ch = -(-E // CHUNK)
    cpt = -(-(-(-ch // NW)) // 16) * 16
    u = 2 * cpt
    stage = 32
    cpt0 = u
    cpt1 = u - cpt0
    passes = cpt0 // stage
    n_chunks = NS * u + stage
    e_pad = n_chunks * CHUNK
    ce0 = cst_edges[0].astype(jnp.int32)
    ce1 = cst_edges[1].astype(jnp.int32)
    oi = ce1 * 2 + LE.astype(jnp.int32) + (num_val - n)
    ii = ce0 + (num_cst - SEG)
    pad = e_pad - E
    oi = jnp.concatenate([oi, jnp.zeros((pad,), jnp.int32)]).reshape(-1, CHUNK)
    ii = jnp.concatenate(
        [ii, jnp.full((pad,), SEG, jnp.int32)]).reshape(-1, CHUNK)

    part = _make_edge_kernel(cpt0, cpt1, stage, passes)(m2, oi, ii)
    r_cst = _combine(part)
    return (r_cst, x_val)


# speedup vs baseline: 3.0737x; 3.0737x over previous
import functools

import jax
import jax.numpy as jnp
from jax import lax
from jax.experimental import pallas as pl
from jax.experimental.pallas import tpu as pltpu
from jax.experimental.pallas import tpu_sc as plsc

HIDDEN = 128
SEG = 10000
NC, NS = 2, 16
NW = NC * NS
CHUNK = 128
SEGP = 10240
ZCH = SEGP // (NS * CHUNK)
ROWS_PER_TILE = SEGP // NS


def _ln(x, g, b, eps=1e-5):
    mu = jnp.mean(x, axis=-1, keepdims=True)
    xc = x - mu
    var = jnp.mean(xc * xc, axis=-1, keepdims=True)
    return xc * lax.rsqrt(var + eps) * g + b


def _mlp_body(h_ref, a_ref, w1t_ref, w1b_ref, b1_ref, w2t_ref, g1_ref,
              bb1_ref, w3t_ref, g2_ref, bb2_ref, x_ref, m_ref):
    h = h_ref[...]
    t = jnp.dot(h, w1t_ref[...], preferred_element_type=jnp.float32)
    t = t + a_ref[...] * w1b_ref[...] + b1_ref[...]
    t = jnp.maximum(t, 0.0)
    x = jnp.dot(t, w2t_ref[...], preferred_element_type=jnp.float32)
    x = _ln(x, g1_ref[...], bb1_ref[...])
    x_ref[...] = x
    m = jnp.dot(x, w3t_ref[...], preferred_element_type=jnp.float32)
    m_ref[...] = _ln(m, g2_ref[...], bb2_ref[...])


def _mlp(h_val, assign, W1, b1, W2, ln1_g, ln1_b, W3, ln2_g, ln2_b):
    n = h_val.shape[0]
    blk = 1000
    grid = n // blk
    full = lambda r, c: pl.BlockSpec((r, c), lambda i: (0, 0))
    return pl.pallas_call(
        _mlp_body,
        grid=(grid,),
        in_specs=[
            pl.BlockSpec((blk, HIDDEN), lambda i: (i, 0)),
            pl.BlockSpec((blk, 1), lambda i: (i, 0)),
            full(HIDDEN, HIDDEN),
            full(1, HIDDEN),
            full(1, HIDDEN),
            full(HIDDEN, HIDDEN),
            full(1, HIDDEN),
            full(1, HIDDEN),
            full(HIDDEN, 2 * HIDDEN),
            full(1, 2 * HIDDEN),
            full(1, 2 * HIDDEN),
        ],
        out_specs=[
            pl.BlockSpec((blk, HIDDEN), lambda i: (i, 0)),
            pl.BlockSpec((blk, 2 * HIDDEN), lambda i: (i, 0)),
        ],
        out_shape=[
            jax.ShapeDtypeStruct((n, HIDDEN), jnp.float32),
            jax.ShapeDtypeStruct((n, 2 * HIDDEN), jnp.float32),
        ],
    )(
        h_val,
        assign.reshape(n, 1),
        W1[:, :HIDDEN].T,
        W1[:, HIDDEN].reshape(1, HIDDEN),
        b1.reshape(1, HIDDEN),
        W2.T,
        ln1_g.reshape(1, HIDDEN),
        ln1_b.reshape(1, HIDDEN),
        W3.T,
        ln2_g.reshape(1, 2 * HIDDEN),
        ln2_b.reshape(1, 2 * HIDDEN),
    )


def _make_edge_kernel(cpt0, cpt1, stage, passes):
    mesh = plsc.VectorSubcoreMesh(
        core_axis_name="c", subcore_axis_name="s", num_cores=NC,
        num_subcores=NS)

    @functools.partial(
        pl.kernel,
        mesh=mesh,
        out_type=jax.ShapeDtypeStruct((NC, SEGP, HIDDEN), jnp.float32),
        scratch_types=[
            pltpu.VMEM((stage, CHUNK), jnp.int32),
            pltpu.VMEM((stage, CHUNK), jnp.int32),
            pltpu.VMEM((2 * CHUNK, HIDDEN), jnp.float32),
            pltpu.VMEM_SHARED((SEGP, HIDDEN), jnp.float32),
            pltpu.SemaphoreType.DMA,
        ],
    )
    def edge_kernel(m_hbm, oi_hbm, ii_hbm, out_hbm, oi_v, ii_v, buf, acc_sh,
                    sg):
        cid = lax.axis_index("c")
        sid = lax.axis_index("s")
        my_cpt = jnp.where(cid == 0, cpt0, cpt1)
        base0 = jnp.where(cid == 0, sid * cpt0, NS * cpt0 + sid * cpt1)

        zeros16 = jnp.zeros((16,), jnp.float32)

        def zrow(i, _):
            for k in range(HIDDEN // 16):
                buf[i, pl.ds(k * 16, 16)] = zeros16
            return 0

        lax.fori_loop(0, 2 * CHUNK, zrow, 0)
        for z in range(ZCH):
            pltpu.sync_copy(
                buf.at[pl.ds(0, CHUNK)],
                acc_sh.at[pl.ds((sid * ZCH + z) * CHUNK, CHUNK)])
        plsc.subcore_barrier()

        def body(j, _):
            @pl.when(j < stage)
            def _():
                pltpu.async_copy(
                    m_hbm.at[oi_v.at[j]],
                    buf.at[pl.ds((j % 2) * CHUNK, CHUNK)], sg)

            @pl.when(j > 0)
            def _():
                jp = j - 1
                slot = jp % 2
                pltpu.make_async_copy(
                    m_hbm.at[oi_v.at[jp]],
                    buf.at[pl.ds(slot * CHUNK, CHUNK)], sg).wait()
                pltpu.sync_copy(
                    buf.at[pl.ds(slot * CHUNK, CHUNK)],
                    acc_sh.at[ii_v.at[jp]], add=True)

            return 0

        for h in range(passes):
            @pl.when(h * stage < my_cpt)
            def _():
                cbase = pl.multiple_of(base0 + h * stage, 8)
                pltpu.sync_copy(oi_hbm.at[pl.ds(cbase, stage)], oi_v)
                pltpu.sync_copy(ii_hbm.at[pl.ds(cbase, stage)], ii_v)
                lax.fori_loop(0, stage + 1, body, 0)
        plsc.subcore_barrier()

        pltpu.sync_copy(
            acc_sh.at[pl.ds(sid * ROWS_PER_TILE, ROWS_PER_TILE)],
            out_hbm.at[cid, pl.ds(sid * ROWS_PER_TILE, ROWS_PER_TILE)])

    return edge_kernel


def _comb_body(p_ref, o_ref):
    o_ref[...] = p_ref[0] + p_ref[1]


def _combine(part):
    blk = 2000
    return pl.pallas_call(
        _comb_body,
        grid=(SEG // blk,),
        in_specs=[pl.BlockSpec((NC, blk, HIDDEN), lambda i: (0, i, 0))],
        out_specs=pl.BlockSpec((blk, HIDDEN), lambda i: (i, 0)),
        out_shape=jax.ShapeDtypeStruct((SEG, HIDDEN), jnp.float32),
    )(part)


def kernel(h_val, assign, cst_edges, LE, num_val, num_cst,
           W1, b1, W2, ln1_g, ln1_b, W3, ln2_g, ln2_b):
    n = h_val.shape[0]
    E = cst_edges.shape[1]

    x_val, m = _mlp(h_val, assign, W1, b1, W2, ln1_g, ln1_b, W3, ln2_g, ln2_b)
    m2 = m.reshape(2 * n, HIDDEN)

    ch = -(-E // CHUNK)
    cpt = -(-(-(-ch // NW)) // 16) * 16
    cpt0 = cpt1 = cpt
    stage = cpt // 2
    passes = cpt0 // stage
    n_chunks = NS * (cpt0 + cpt1) + stage
    e_pad = n_chunks * CHUNK
    ce0 = cst_edges[0].astype(jnp.int32)
    ce1 = cst_edges[1].astype(jnp.int32)
    oi = ce1 * 2 + LE.astype(jnp.int32) + (num_val - n)
    ii = ce0 + (num_cst - SEG)
    pad = e_pad - E
    spread = jnp.arange(pad, dtype=jnp.int32)
    oi = jnp.concatenate([oi, spread % (2 * n)]).reshape(-1, CHUNK)
    ii = jnp.concatenate(
        [ii, SEG + spread % (SEGP - SEG)]).reshape(-1, CHUNK)

    part = _make_edge_kernel(cpt0, cpt1, stage, passes)(m2, oi, ii)
    r_cst = _combine(part)
    return (r_cst, x_val)


# speedup vs baseline: 3.3286x; 1.0829x over previous
import functools

import jax
import jax.numpy as jnp
from jax import lax
from jax.experimental import pallas as pl
from jax.experimental.pallas import tpu as pltpu
from jax.experimental.pallas import tpu_sc as plsc

HIDDEN = 128
SEG = 10000
NC, NS = 2, 16
NW = NC * NS
CHUNK = 128
SEGP = 10240
ZCH = SEGP // (NS * CHUNK)
ROWS_PER_TILE = SEGP // NS


def _ln(x, g, b, eps=1e-5):
    mu = jnp.mean(x, axis=-1, keepdims=True)
    xc = x - mu
    var = jnp.mean(xc * xc, axis=-1, keepdims=True)
    return xc * lax.rsqrt(var + eps) * g + b


_DN = (((1,), (1,)), ((), ()))


def _mlp_body(h_ref, a_ref, w1_ref, w1b_ref, b1_ref, w2_ref, g1_ref,
              bb1_ref, w3_ref, g2_ref, bb2_ref, x_ref, m_ref):
    h = h_ref[...]
    t = lax.dot_general(h, w1_ref[...], _DN,
                        preferred_element_type=jnp.float32)
    t = t + a_ref[...] * w1b_ref[...] + b1_ref[...]
    t = jnp.maximum(t, 0.0)
    x = lax.dot_general(t, w2_ref[...], _DN,
                        preferred_element_type=jnp.float32)
    x = _ln(x, g1_ref[...], bb1_ref[...])
    x_ref[...] = x
    m = lax.dot_general(x, w3_ref[...], _DN,
                        preferred_element_type=jnp.float32)
    m = _ln(m, g2_ref[...], bb2_ref[...])
    m_ref[...] = m.reshape(m_ref.shape)


def _mlp(h_val, assign, W1, b1, W2, ln1_g, ln1_b, W3, ln2_g, ln2_b):
    n = h_val.shape[0]
    blk = 1000
    grid = n // blk
    full = lambda r, c: pl.BlockSpec((r, c), lambda i: (0, 0))
    return pl.pallas_call(
        _mlp_body,
        grid=(grid,),
        in_specs=[
            pl.BlockSpec((blk, HIDDEN), lambda i: (i, 0)),
            pl.BlockSpec((blk, 1), lambda i: (i, 0)),
            full(HIDDEN, HIDDEN),
            full(1, HIDDEN),
            full(1, HIDDEN),
            full(HIDDEN, HIDDEN),
            full(1, HIDDEN),
            full(1, HIDDEN),
            full(2 * HIDDEN, HIDDEN),
            full(1, 2 * HIDDEN),
            full(1, 2 * HIDDEN),
        ],
        out_specs=[
            pl.BlockSpec((blk, HIDDEN), lambda i: (i, 0)),
            pl.BlockSpec((2 * blk, HIDDEN), lambda i: (i, 0)),
        ],
        out_shape=[
            jax.ShapeDtypeStruct((n, HIDDEN), jnp.float32),
            jax.ShapeDtypeStruct((2 * n, HIDDEN), jnp.float32),
        ],
    )(
        h_val,
        assign.reshape(n, 1),
        W1[:, :HIDDEN],
        W1[:, HIDDEN].reshape(1, HIDDEN),
        b1.reshape(1, HIDDEN),
        W2,
        ln1_g.reshape(1, HIDDEN),
        ln1_b.reshape(1, HIDDEN),
        W3,
        ln2_g.reshape(1, 2 * HIDDEN),
        ln2_b.reshape(1, 2 * HIDDEN),
    )


def _make_edge_kernel(cpt0, cpt1, stage, passes):
    mesh = plsc.VectorSubcoreMesh(
        core_axis_name="c", subcore_axis_name="s", num_cores=NC,
        num_subcores=NS)

    @functools.partial(
        pl.kernel,
        mesh=mesh,
        out_type=jax.ShapeDtypeStruct((NC, SEGP, HIDDEN), jnp.float32),
        scratch_types=[
            pltpu.VMEM((stage, CHUNK), jnp.int32),
            pltpu.VMEM((stage, CHUNK), jnp.int32),
            pltpu.VMEM((2 * CHUNK, HIDDEN), jnp.float32),
            pltpu.VMEM_SHARED((SEGP, HIDDEN), jnp.float32),
            pltpu.SemaphoreType.DMA,
        ],
    )
    def edge_kernel(m_hbm, oi_hbm, ii_hbm, out_hbm, oi_v, ii_v, buf, acc_sh,
                    sg):
        cid = lax.axis_index("c")
        sid = lax.axis_index("s")
        my_cpt = jnp.where(cid == 0, cpt0, cpt1)
        base0 = jnp.where(cid == 0, sid * cpt0, NS * cpt0 + sid * cpt1)

        zeros16 = jnp.zeros((16,), jnp.float32)

        def zrow(i, _):
            for k in range(HIDDEN // 16):
                buf[i, pl.ds(k * 16, 16)] = zeros16
            return 0

        lax.fori_loop(0, 2 * CHUNK, zrow, 0)
        for z in range(ZCH):
            pltpu.sync_copy(
                buf.at[pl.ds(0, CHUNK)],
                acc_sh.at[pl.ds((sid * ZCH + z) * CHUNK, CHUNK)])
        plsc.subcore_barrier()

        def body(j, _):
            @pl.when(j < stage)
            def _():
                pltpu.async_copy(
                    m_hbm.at[oi_v.at[j]],
                    buf.at[pl.ds((j % 2) * CHUNK, CHUNK)], sg)

            @pl.when(j > 0)
            def _():
                jp = j - 1
                slot = jp % 2
                pltpu.make_async_copy(
                    m_hbm.at[oi_v.at[jp]],
                    buf.at[pl.ds(slot * CHUNK, CHUNK)], sg).wait()
                pltpu.sync_copy(
                    buf.at[pl.ds(slot * CHUNK, CHUNK)],
                    acc_sh.at[ii_v.at[jp]], add=True)

            return 0

        for h in range(passes):
            @pl.when(h * stage < my_cpt)
            def _():
                cbase = pl.multiple_of(base0 + h * stage, 8)
                pltpu.sync_copy(oi_hbm.at[pl.ds(cbase, stage)], oi_v)
                pltpu.sync_copy(ii_hbm.at[pl.ds(cbase, stage)], ii_v)
                lax.fori_loop(0, stage + 1, body, 0)
        plsc.subcore_barrier()

        pltpu.sync_copy(
            acc_sh.at[pl.ds(sid * ROWS_PER_TILE, ROWS_PER_TILE)],
            out_hbm.at[cid, pl.ds(sid * ROWS_PER_TILE, ROWS_PER_TILE)])

    return edge_kernel


def _comb_body(p_ref, o_ref):
    o_ref[...] = p_ref[0] + p_ref[1]


def _combine(part):
    blk = 2000
    return pl.pallas_call(
        _comb_body,
        grid=(SEG // blk,),
        in_specs=[pl.BlockSpec((NC, blk, HIDDEN), lambda i: (0, i, 0))],
        out_specs=pl.BlockSpec((blk, HIDDEN), lambda i: (i, 0)),
        out_shape=jax.ShapeDtypeStruct((SEG, HIDDEN), jnp.float32),
    )(part)


def kernel(h_val, assign, cst_edges, LE, num_val, num_cst,
           W1, b1, W2, ln1_g, ln1_b, W3, ln2_g, ln2_b):
    n = h_val.shape[0]
    E = cst_edges.shape[1]

    x_val, m2 = _mlp(h_val, assign, W1, b1, W2, ln1_g, ln1_b, W3, ln2_g, ln2_b)

    ch = -(-E // CHUNK)
    cpt = -(-(-(-ch // NW)) // 16) * 16
    cpt0 = cpt1 = cpt
    stage = cpt // 2
    passes = cpt0 // stage
    n_chunks = NS * (cpt0 + cpt1) + stage
    e_pad = n_chunks * CHUNK
    ce0 = cst_edges[0].astype(jnp.int32)
    ce1 = cst_edges[1].astype(jnp.int32)
    oi = ce1 * 2 + LE.astype(jnp.int32) + (num_val - n)
    ii = ce0 + (num_cst - SEG)
    pad = e_pad - E
    spread = jnp.arange(pad, dtype=jnp.int32)
    oi = jnp.concatenate([oi, spread % (2 * n)]).reshape(-1, CHUNK)
    ii = jnp.concatenate(
        [ii, SEG + spread % (SEGP - SEG)]).reshape(-1, CHUNK)

    part = _make_edge_kernel(cpt0, cpt1, stage, passes)(m2, oi, ii)
    r_cst = _combine(part)
    return (r_cst, x_val)


# speedup vs baseline: 3.5863x; 1.0774x over previous
import functools

import jax
import jax.numpy as jnp
from jax import lax
from jax.experimental import pallas as pl
from jax.experimental.pallas import tpu as pltpu
from jax.experimental.pallas import tpu_sc as plsc

HIDDEN = 128
SEG = 10000
NC, NS = 2, 16
NW = NC * NS
CHUNK = 128
SEGP = 10240
ZCH = SEGP // (NS * CHUNK)
ROWS_PER_TILE = SEGP // NS


def _ln(x, g, b, eps=1e-5):
    mu = jnp.mean(x, axis=-1, keepdims=True)
    xc = x - mu
    var = jnp.mean(xc * xc, axis=-1, keepdims=True)
    return xc * lax.rsqrt(var + eps) * g + b


_DN = (((1,), (1,)), ((), ()))


def _make_mlp_body(n, E, r_blk):
    eblk = r_blk * CHUNK

    def body(h_ref, a_ref, ce_ref, le_ref, off_ref, w1_ref, w1b_ref, b1_ref,
             w2_ref, g1_ref, bb1_ref, w3_ref, g2_ref, bb2_ref,
             x_ref, m_ref, oi_ref, ii_ref):
        h = h_ref[...]
        t = lax.dot_general(h, w1_ref[...], _DN,
                            preferred_element_type=jnp.float32)
        t = t + a_ref[...] * w1b_ref[...] + b1_ref[...]
        t = jnp.maximum(t, 0.0)
        x = lax.dot_general(t, w2_ref[...], _DN,
                            preferred_element_type=jnp.float32)
        x = _ln(x, g1_ref[...], bb1_ref[...])
        x_ref[...] = x
        m = lax.dot_general(x, w3_ref[...], _DN,
                            preferred_element_type=jnp.float32)
        m = _ln(m, g2_ref[...], bb2_ref[...])
        m_ref[...] = m.reshape(m_ref.shape)

        ce0 = ce_ref[0, :].reshape(r_blk, CHUNK)
        ce1 = ce_ref[1, :].reshape(r_blk, CHUNK)
        le = le_ref[...].reshape(r_blk, CHUNK)
        off = off_ref[...]
        eid = (pl.program_id(0) * eblk
               + lax.broadcasted_iota(jnp.int32, (r_blk, CHUNK), 0) * CHUNK
               + lax.broadcasted_iota(jnp.int32, (r_blk, CHUNK), 1))
        pad = eid >= E
        oi_ref[...] = jnp.where(pad, eid % (2 * n), ce1 * 2 + le + off[0, 0])
        ii_ref[...] = jnp.where(pad, SEG + eid % (SEGP - SEG), ce0 + off[0, 1])

    return body


def _mlp(h_val, assign, cst_edges, LE, offs, n_chunks,
         W1, b1, W2, ln1_g, ln1_b, W3, ln2_g, ln2_b):
    n = h_val.shape[0]
    E = cst_edges.shape[1]
    blk = 1000
    grid = n // blk
    r_blk = -(-(-(-n_chunks // grid)) // 8) * 8
    rows = r_blk * grid
    eblk = r_blk * CHUNK
    full = lambda r, c: pl.BlockSpec((r, c), lambda i: (0, 0))
    return pl.pallas_call(
        _make_mlp_body(n, E, r_blk),
        grid=(grid,),
        in_specs=[
            pl.BlockSpec((blk, HIDDEN), lambda i: (i, 0)),
            pl.BlockSpec((blk, 1), lambda i: (i, 0)),
            pl.BlockSpec((2, eblk), lambda i: (0, i)),
            pl.BlockSpec((eblk,), lambda i: (i,)),
            full(1, 2),
            full(HIDDEN, HIDDEN),
            full(1, HIDDEN),
            full(1, HIDDEN),
            full(HIDDEN, HIDDEN),
            full(1, HIDDEN),
            full(1, HIDDEN),
            full(2 * HIDDEN, HIDDEN),
            full(1, 2 * HIDDEN),
            full(1, 2 * HIDDEN),
        ],
        out_specs=[
            pl.BlockSpec((blk, HIDDEN), lambda i: (i, 0)),
            pl.BlockSpec((2 * blk, HIDDEN), lambda i: (i, 0)),
            pl.BlockSpec((r_blk, CHUNK), lambda i: (i, 0)),
            pl.BlockSpec((r_blk, CHUNK), lambda i: (i, 0)),
        ],
        out_shape=[
            jax.ShapeDtypeStruct((n, HIDDEN), jnp.float32),
            jax.ShapeDtypeStruct((2 * n, HIDDEN), jnp.float32),
            jax.ShapeDtypeStruct((rows, CHUNK), jnp.int32),
            jax.ShapeDtypeStruct((rows, CHUNK), jnp.int32),
        ],
    )(
        h_val,
        assign.reshape(n, 1),
        cst_edges,
        LE,
        offs,
        W1[:, :HIDDEN],
        W1[:, HIDDEN].reshape(1, HIDDEN),
        b1.reshape(1, HIDDEN),
        W2,
        ln1_g.reshape(1, HIDDEN),
        ln1_b.reshape(1, HIDDEN),
        W3,
        ln2_g.reshape(1, 2 * HIDDEN),
        ln2_b.reshape(1, 2 * HIDDEN),
    )


def _make_edge_kernel(cpt0, cpt1, stage, passes):
    mesh = plsc.VectorSubcoreMesh(
        core_axis_name="c", subcore_axis_name="s", num_cores=NC,
        num_subcores=NS)

    @functools.partial(
        pl.kernel,
        mesh=mesh,
        out_type=jax.ShapeDtypeStruct((NC, SEGP, HIDDEN), jnp.float32),
        scratch_types=[
            pltpu.VMEM((stage, CHUNK), jnp.int32),
            pltpu.VMEM((stage, CHUNK), jnp.int32),
            pltpu.VMEM((2 * CHUNK, HIDDEN), jnp.float32),
            pltpu.VMEM_SHARED((SEGP, HIDDEN), jnp.float32),
            pltpu.SemaphoreType.DMA,
        ],
    )
    def edge_kernel(m_hbm, oi_hbm, ii_hbm, out_hbm, oi_v, ii_v, buf, acc_sh,
                    sg):
        cid = lax.axis_index("c")
        sid = lax.axis_index("s")
        my_cpt = jnp.where(cid == 0, cpt0, cpt1)
        base0 = jnp.where(cid == 0, sid * cpt0, NS * cpt0 + sid * cpt1)

        zeros16 = jnp.zeros((16,), jnp.float32)

        def zrow(i, _):
            for k in range(HIDDEN // 16):
                buf[i, pl.ds(k * 16, 16)] = zeros16
            return 0

        lax.fori_loop(0, 2 * CHUNK, zrow, 0)
        for z in range(ZCH):
            pltpu.sync_copy(
                buf.at[pl.ds(0, CHUNK)],
                acc_sh.at[pl.ds((sid * ZCH + z) * CHUNK, CHUNK)])
        plsc.subcore_barrier()

        def body(j, _):
            @pl.when(j < stage)
            def _():
                pltpu.async_copy(
                    m_hbm.at[oi_v.at[j]],
                    buf.at[pl.ds((j % 2) * CHUNK, CHUNK)], sg)

            @pl.when(j > 0)
            def _():
                jp = j - 1
                slot = jp % 2
                pltpu.make_async_copy(
                    m_hbm.at[oi_v.at[jp]],
                    buf.at[pl.ds(slot * CHUNK, CHUNK)], sg).wait()
                pltpu.sync_copy(
                    buf.at[pl.ds(slot * CHUNK, CHUNK)],
                    acc_sh.at[ii_v.at[jp]], add=True)

            return 0

        for h in range(passes):
            @pl.when(h * stage < my_cpt)
            def _():
                cbase = pl.multiple_of(base0 + h * stage, 8)
                pltpu.sync_copy(oi_hbm.at[pl.ds(cbase, stage)], oi_v)
                pltpu.sync_copy(ii_hbm.at[pl.ds(cbase, stage)], ii_v)
                lax.fori_loop(0, stage + 1, body, 0)
        plsc.subcore_barrier()

        pltpu.sync_copy(
            acc_sh.at[pl.ds(sid * ROWS_PER_TILE, ROWS_PER_TILE)],
            out_hbm.at[cid, pl.ds(sid * ROWS_PER_TILE, ROWS_PER_TILE)])

    return edge_kernel


def _comb_body(p_ref, o_ref):
    o_ref[...] = p_ref[0] + p_ref[1]


def _combine(part):
    blk = 2000
    return pl.pallas_call(
        _comb_body,
        grid=(SEG // blk,),
        in_specs=[pl.BlockSpec((NC, blk, HIDDEN), lambda i: (0, i, 0))],
        out_specs=pl.BlockSpec((blk, HIDDEN), lambda i: (i, 0)),
        out_shape=jax.ShapeDtypeStruct((SEG, HIDDEN), jnp.float32),
    )(part)


def kernel(h_val, assign, cst_edges, LE, num_val, num_cst,
           W1, b1, W2, ln1_g, ln1_b, W3, ln2_g, ln2_b):
    n = h_val.shape[0]
    E = cst_edges.shape[1]

    ch = -(-E // CHUNK)
    cpt = -(-(-(-ch // NW)) // 16) * 16
    cpt0 = cpt1 = cpt
    stage = cpt // 2
    passes = cpt0 // stage
    n_chunks = NS * (cpt0 + cpt1) + stage
    offs = jnp.stack([num_val - n, num_cst - SEG]).astype(
        jnp.int32).reshape(1, 2)
    x_val, m2, oi, ii = _mlp(
        h_val, assign, cst_edges.astype(jnp.int32), LE.astype(jnp.int32),
        offs, n_chunks, W1, b1, W2, ln1_g, ln1_b, W3, ln2_g, ln2_b)

    part = _make_edge_kernel(cpt0, cpt1, stage, passes)(m2, oi, ii)
    r_cst = _combine(part)
    return (r_cst, x_val)
